# baseline (speedup 1.0000x reference)
import functools

import jax
import jax.numpy as jnp
from jax import lax
from jax.experimental import pallas as pl
from jax.experimental.pallas import tpu as pltpu
from jax.experimental.pallas import tpu_sc as plsc

_B, _S, _D = 4, 4096, 768
_N = _B * _S
_NC, _NS = 2, 16
_NW = _NC * _NS
_SPW = _S // _NW
_CH = 32
_NSUB = _SPW // _CH
_NCHUNK = _NSUB * _B
_LANES = 16
_JV = _D // _LANES


def _make_emb_kernel():
    mesh = plsc.VectorSubcoreMesh(core_axis_name="c", subcore_axis_name="s")

    @functools.partial(
        pl.kernel,
        mesh=mesh,
        out_type=jax.ShapeDtypeStruct((_N, _D), jnp.float32),
        scratch_types=[
            pltpu.VMEM((_B, _SPW), jnp.int32),
            pltpu.VMEM((_B, _CH, _D), jnp.float32),
            pltpu.VMEM((_CH, _D), jnp.float32),
            pltpu.SemaphoreType.DMA,
            pltpu.SemaphoreType.DMA,
            pltpu.SemaphoreType.DMA,
            pltpu.SemaphoreType.DMA,
            pltpu.SemaphoreType.DMA,
            pltpu.SemaphoreType.DMA,
            pltpu.SemaphoreType.DMA,
            pltpu.SemaphoreType.DMA,
            pltpu.SemaphoreType.DMA,
        ],
    )
    def emb(x_hbm, table_hbm, pe_hbm, out_hbm,
            idx_v, rows_v, pe_v, sem_i,
            sem_g0, sem_g1, sem_g2, sem_g3,
            sem_o0, sem_o1, sem_o2, sem_o3):
        wid = lax.axis_index("s") * _NC + lax.axis_index("c")
        s_base = wid * _SPW
        sems_g = (sem_g0, sem_g1, sem_g2, sem_g3)
        sems_o = (sem_o0, sem_o1, sem_o2, sem_o3)

        pltpu.make_async_copy(
            x_hbm.at[:, pl.ds(s_base, _SPW)], idx_v, sem_i).start()

        def gather_desc(sub, b):
            idx_sl = idx_v.at[b, pl.ds(sub * _CH, _CH)]
            return pltpu.make_async_copy(table_hbm.at[idx_sl],
                                         rows_v.at[b], sems_g[b])

        def out_desc(sub, b):
            row0 = b * _S + s_base + sub * _CH
            return pltpu.make_async_copy(rows_v.at[b],
                                         out_hbm.at[pl.ds(row0, _CH)],
                                         sems_o[b])

        pltpu.make_async_copy(
            x_hbm.at[:, pl.ds(s_base, _SPW)], idx_v, sem_i).wait()

        @pl.loop(0, _NSUB)
        def _subs(sub):
            pltpu.async_copy(pe_hbm.at[pl.ds(s_base + sub * _CH, _CH)],
                             pe_v, sem_i)
            for b in range(_B):
                @pl.when(sub > 0)
                def _():
                    out_desc(sub - 1, b).wait()
                gather_desc(sub, b).start()
            pltpu.make_async_copy(
                pe_hbm.at[pl.ds(s_base + sub * _CH, _CH)], pe_v, sem_i).wait()
            for b in range(_B):
                gather_desc(sub, b).wait()

                def row_body(r, carry):
                    for j in range(_JV):
                        sl = pl.ds(j * _LANES, _LANES)
                        plsc.addupdate(rows_v.at[b, r, sl], pe_v[r, sl])
                    return carry

                lax.fori_loop(0, _CH, row_body, 0)
                out_desc(sub, b).start()

        for b in range(_B):
            out_desc(_NSUB - 1, b).wait()

    return emb


_emb = _make_emb_kernel()


def kernel(x, table, pe):
    out = _emb(x.astype(jnp.int32), table, pe)
    return out.reshape(_B, _S, _D)


# speedup vs baseline: 1.1000x; 1.1000x over previous
import functools

import jax
import jax.numpy as jnp
from jax import lax
from jax.experimental import pallas as pl
from jax.experimental.pallas import tpu as pltpu
from jax.experimental.pallas import tpu_sc as plsc

_B, _S, _D = 4, 4096, 768
_N = _B * _S
_NC, _NS = 2, 16
_NW = _NC * _NS
_SPW = _S // _NW
_CH = 32
_NSUB = _SPW // _CH
_NCHUNK = _NSUB * _B
_LANES = 16
_JV = _D // _LANES


def _make_emb_kernel():
    mesh = plsc.VectorSubcoreMesh(core_axis_name="c", subcore_axis_name="s")

    @functools.partial(
        pl.kernel,
        mesh=mesh,
        out_type=jax.ShapeDtypeStruct((_N, _D), jnp.float32),
        scratch_types=[
            pltpu.VMEM((_B, _SPW), jnp.int32),
            pltpu.VMEM((2, _CH, _D), jnp.float32),
            pltpu.VMEM((2, _CH, _D), jnp.float32),
            pltpu.SemaphoreType.DMA,
            pltpu.SemaphoreType.DMA,
            pltpu.SemaphoreType.DMA,
            pltpu.SemaphoreType.DMA,
            pltpu.SemaphoreType.DMA,
            pltpu.SemaphoreType.DMA,
            pltpu.SemaphoreType.DMA,
        ],
    )
    def emb(x_hbm, table_hbm, pe_hbm, out_hbm,
            idx_v, rows_v, pe_v, sem_i,
            sem_g0, sem_g1, sem_o0, sem_o1, sem_p0, sem_p1):
        wid = lax.axis_index("s") * _NC + lax.axis_index("c")
        s_base = wid * _SPW
        sems_g = (sem_g0, sem_g1)
        sems_o = (sem_o0, sem_o1)
        sems_p = (sem_p0, sem_p1)

        def coords(t):
            sub = t // _B
            b = t % _B
            row0 = b * _S + s_base + sub * _CH
            return sub, b, row0

        def pe_desc(sub, p):
            return pltpu.make_async_copy(
                pe_hbm.at[pl.ds(s_base + sub * _CH, _CH)],
                pe_v.at[p], sems_p[p])

        def gather_desc(t, k):
            sub, b, _ = coords(t)
            idx_sl = idx_v.at[b, pl.ds(sub * _CH, _CH)]
            return pltpu.make_async_copy(table_hbm.at[idx_sl],
                                         rows_v.at[k], sems_g[k])

        def out_desc(t, k):
            _, _, row0 = coords(t)
            return pltpu.make_async_copy(rows_v.at[k],
                                         out_hbm.at[pl.ds(row0, _CH)],
                                         sems_o[k])

        idx_desc = pltpu.make_async_copy(
            x_hbm.at[:, pl.ds(s_base, _SPW)], idx_v, sem_i)
        idx_desc.start()
        pe_desc(0, 0).start()
        pe_desc(1, 1).start()
        idx_desc.wait()

        for sub in range(_NSUB):
            p = sub % 2
            pe_desc(sub, p).wait()

            @pl.loop(0, _B, step=2)
            def _pair(c):
                t0 = sub * _B + c
                for k in range(2):
                    t = t0 + k
                    if sub == 0:

                        @pl.when(c > 0)
                        def _():
                            out_desc(lax.max(t - 2, 0), k).wait()
                    else:
                        out_desc(t - 2, k).wait()
                    gather_desc(t, k).start()
                for k in range(2):
                    t = t0 + k
                    gather_desc(t, k).wait()

                    def row_body(r, carry):
                        for j in range(_JV):
                            sl = pl.ds(j * _LANES, _LANES)
                            plsc.addupdate(rows_v.at[k, r, sl], pe_v[p, r, sl])
                        return carry

                    lax.fori_loop(0, _CH, row_body, 0)
                    out_desc(t, k).start()

            if sub + 2 < _NSUB:
                pe_desc(sub + 2, p).start()

        for k in range(2):
            out_desc(_NCHUNK - 2 + k, k).wait()

    return emb


_emb = _make_emb_kernel()


def kernel(x, table, pe):
    out = _emb(x.astype(jnp.int32), table, pe)
    return out.reshape(_B, _S, _D)
